# baseline (speedup 1.0000x reference)
import jax
import jax.numpy as jnp
from jax import lax
from jax.experimental import pallas as pl
from jax.experimental.pallas import tpu as pltpu
from jax.experimental.pallas import tpu_sc as plsc

VOCAB = 100000
EMB = 512
B = 16384
L = 20
H1 = 256

NC = 2
NS = 16
NW = NC * NS
BPW = B // NW
CB = 4
CHUNK = CB * L
CB_OUT = 2 * CB
N_OUT = BPW // CB_OUT
LANES = 16
NCOL = H1 // LANES


def _project_body(t_ref, w1_ref, o_ref):
    dn = (((1,), (1,)), ((), ()))
    o_ref[...] = lax.dot_general(t_ref[...], w1_ref[...], dn,
                                 preferred_element_type=jnp.float32)


def _project(table, W1):
    RB = 1000
    return pl.pallas_call(
        _project_body,
        grid=(VOCAB // RB,),
        in_specs=[
            pl.BlockSpec((RB, EMB), lambda i: (i, 0)),
            pl.BlockSpec((H1, EMB), lambda i: (0, 0)),
        ],
        out_specs=pl.BlockSpec((RB, H1), lambda i: (i, 0)),
        out_shape=jax.ShapeDtypeStruct((VOCAB, H1), jnp.float32),
    )(table, W1)


def _sc_pool_body(table_hbm, tok_hbm, out_hbm,
                  idx0, idx1, rows0, rows1, stage_v,
                  sem0, sem1):
    wid = lax.axis_index("s") * NC + lax.axis_index("c")
    idx_base = wid * (BPW * L)
    out_base = wid * BPW

    idx_bufs = [idx0, idx1]
    row_bufs = [rows0, rows1]
    sems = [sem0, sem1]

    def gather_start(g, buf):
        pltpu.sync_copy(tok_hbm.at[pl.ds(idx_base + g * CHUNK, CHUNK)],
                        idx_bufs[buf])
        pltpu.async_copy(table_hbm.at[idx_bufs[buf]], row_bufs[buf],
                         sems[buf])

    def gather_wait(buf):
        pltpu.make_async_copy(table_hbm.at[idx_bufs[buf]], row_bufs[buf],
                              sems[buf]).wait()

    def pool_chunk(rows, stage_base):
        def col_body(c, carry):
            sl = pl.ds(c * LANES, LANES)
            for b in range(CB):
                s0 = rows[b * L + 0, sl]
                s1 = rows[b * L + 1, sl]
                for r in range(2, L, 2):
                    s0 = s0 + rows[b * L + r, sl]
                    s1 = s1 + rows[b * L + r + 1, sl]
                stage_v[stage_base + b, sl] = s0 + s1
            return carry

        lax.fori_loop(0, NCOL, col_body, 0)

    gather_start(0, 0)
    gather_start(1, 1)

    def iter_body(i, carry):
        gather_wait(0)
        pool_chunk(rows0, 0)

        @pl.when(i < N_OUT - 1)
        def _():
            gather_start(2 * i + 2, 0)

        gather_wait(1)
        pool_chunk(rows1, CB)

        @pl.when(i < N_OUT - 1)
        def _():
            gather_start(2 * i + 3, 1)

        pltpu.sync_copy(stage_v,
                        out_hbm.at[pl.ds(out_base + i * CB_OUT, CB_OUT)])
        return carry

    lax.fori_loop(0, N_OUT, iter_body, 0)


def _sc_pool(proj, tokens_flat):
    pool = pl.kernel(
        _sc_pool_body,
        out_type=jax.ShapeDtypeStruct((B, H1), jnp.float32),
        mesh=plsc.VectorSubcoreMesh(core_axis_name="c", subcore_axis_name="s"),
        scratch_types=[
            pltpu.VMEM((CHUNK,), jnp.int32),
            pltpu.VMEM((CHUNK,), jnp.int32),
            pltpu.VMEM((CHUNK, H1), jnp.float32),
            pltpu.VMEM((CHUNK, H1), jnp.float32),
            pltpu.VMEM((CB_OUT, H1), jnp.float32),
            pltpu.SemaphoreType.DMA,
            pltpu.SemaphoreType.DMA,
        ],
    )
    return pool(proj, tokens_flat)


def _head_body(x_ref, b1_ref, w2_ref, b2_ref, w3_ref, b3_ref, o_ref):
    dn = (((1,), (1,)), ((), ()))
    h = jnp.maximum(x_ref[...] * (1.0 / L) + b1_ref[...], 0.0)
    h = lax.dot_general(h, w2_ref[...], dn, preferred_element_type=jnp.float32)
    h = jnp.maximum(h + b2_ref[...], 0.0)
    o = lax.dot_general(h, w3_ref[...], dn, preferred_element_type=jnp.float32)
    o_ref[...] = o + b3_ref[...]


def _head(pooled, b1, W2, b2, W3, b3):
    TB = 1024
    return pl.pallas_call(
        _head_body,
        grid=(B // TB,),
        in_specs=[
            pl.BlockSpec((TB, H1), lambda i: (i, 0)),
            pl.BlockSpec((1, H1), lambda i: (0, 0)),
            pl.BlockSpec((128, H1), lambda i: (0, 0)),
            pl.BlockSpec((1, 128), lambda i: (0, 0)),
            pl.BlockSpec((64, 128), lambda i: (0, 0)),
            pl.BlockSpec((1, 64), lambda i: (0, 0)),
        ],
        out_specs=pl.BlockSpec((TB, 64), lambda i: (i, 0)),
        out_shape=jax.ShapeDtypeStruct((B, 64), jnp.float32),
    )(pooled, b1.reshape(1, -1), W2, b2.reshape(1, -1), W3,
      b3.reshape(1, -1))


def kernel(semantic_tokens, table, W1, b1, W2, b2, W3, b3):
    tokens_flat = semantic_tokens.reshape(-1)
    proj = _project(table, W1)
    pooled_sum = _sc_pool(proj, tokens_flat)
    return _head(pooled_sum, b1, W2, b2, W3, b3)


# speedup vs baseline: 1.1014x; 1.1014x over previous
import jax
import jax.numpy as jnp
from jax import lax
from jax.experimental import pallas as pl
from jax.experimental.pallas import tpu as pltpu
from jax.experimental.pallas import tpu_sc as plsc

VOCAB = 100000
EMB = 512
B = 16384
L = 20
H1 = 256

NC = 2
NS = 16
NW = NC * NS
BPW = B // NW
CB = 4
CHUNK = CB * L
CB_OUT = 2 * CB
N_OUT = BPW // CB_OUT
N_CHUNKS = BPW // CB
LANES = 16
NCOL = H1 // LANES


def _project_body(t_ref, w1_ref, o_ref):
    dn = (((1,), (1,)), ((), ()))
    o_ref[...] = lax.dot_general(t_ref[...], w1_ref[...], dn,
                                 preferred_element_type=jnp.float32)


def _project(table, W1):
    RB = 1000
    return pl.pallas_call(
        _project_body,
        grid=(VOCAB // RB,),
        in_specs=[
            pl.BlockSpec((RB, EMB), lambda i: (i, 0)),
            pl.BlockSpec((H1, EMB), lambda i: (0, 0)),
        ],
        out_specs=pl.BlockSpec((RB, H1), lambda i: (i, 0)),
        out_shape=jax.ShapeDtypeStruct((VOCAB, H1), jnp.float32),
    )(table, W1)


def _sc_pool_body(table_hbm, tok_hbm, out_hbm,
                  idx_all, rows0, rows1, stage_v,
                  sem0, sem1):
    wid = lax.axis_index("s") * NC + lax.axis_index("c")
    out_base = wid * BPW

    row_bufs = [rows0, rows1]
    sems = [sem0, sem1]

    pltpu.sync_copy(tok_hbm.at[pl.ds(wid * N_CHUNKS, N_CHUNKS)], idx_all)

    def gather_start(g, buf):
        pltpu.async_copy(table_hbm.at[idx_all.at[g]], row_bufs[buf],
                         sems[buf])

    def gather_wait(g, buf):
        pltpu.make_async_copy(table_hbm.at[idx_all.at[g]], row_bufs[buf],
                              sems[buf]).wait()

    def pool_chunk(rows, stage_base):
        def col_body(c, carry):
            sl = pl.ds(c * LANES, LANES)
            for b in range(CB):
                s0 = rows[b * L + 0, sl]
                s1 = rows[b * L + 1, sl]
                for r in range(2, L, 2):
                    s0 = s0 + rows[b * L + r, sl]
                    s1 = s1 + rows[b * L + r + 1, sl]
                stage_v[stage_base + b, sl] = s0 + s1
            return carry

        lax.fori_loop(0, NCOL, col_body, 0)

    gather_start(0, 0)
    gather_start(1, 1)

    def iter_body(i, carry):
        gather_wait(2 * i, 0)
        pool_chunk(rows0, 0)

        @pl.when(i < N_OUT - 1)
        def _():
            gather_start(2 * i + 2, 0)

        gather_wait(2 * i + 1, 1)
        pool_chunk(rows1, CB)

        @pl.when(i < N_OUT - 1)
        def _():
            gather_start(2 * i + 3, 1)

        pltpu.sync_copy(stage_v,
                        out_hbm.at[pl.ds(out_base + i * CB_OUT, CB_OUT)])
        return carry

    lax.fori_loop(0, N_OUT, iter_body, 0)


def _sc_pool(proj, tokens_flat):
    pool = pl.kernel(
        _sc_pool_body,
        out_type=jax.ShapeDtypeStruct((B, H1), jnp.float32),
        mesh=plsc.VectorSubcoreMesh(core_axis_name="c", subcore_axis_name="s"),
        scratch_types=[
            pltpu.VMEM((N_CHUNKS, CHUNK), jnp.int32),
            pltpu.VMEM((CHUNK, H1), jnp.float32),
            pltpu.VMEM((CHUNK, H1), jnp.float32),
            pltpu.VMEM((CB_OUT, H1), jnp.float32),
            pltpu.SemaphoreType.DMA,
            pltpu.SemaphoreType.DMA,
        ],
    )
    return pool(proj, tokens_flat)


def _head_body(x_ref, b1_ref, w2_ref, b2_ref, w3_ref, b3_ref, o_ref):
    dn = (((1,), (1,)), ((), ()))
    h = jnp.maximum(x_ref[...] * (1.0 / L) + b1_ref[...], 0.0)
    h = lax.dot_general(h, w2_ref[...], dn, preferred_element_type=jnp.float32)
    h = jnp.maximum(h + b2_ref[...], 0.0)
    o = lax.dot_general(h, w3_ref[...], dn, preferred_element_type=jnp.float32)
    o_ref[...] = o + b3_ref[...]


def _head(pooled, b1, W2, b2, W3, b3):
    TB = 1024
    return pl.pallas_call(
        _head_body,
        grid=(B // TB,),
        in_specs=[
            pl.BlockSpec((TB, H1), lambda i: (i, 0)),
            pl.BlockSpec((1, H1), lambda i: (0, 0)),
            pl.BlockSpec((128, H1), lambda i: (0, 0)),
            pl.BlockSpec((1, 128), lambda i: (0, 0)),
            pl.BlockSpec((64, 128), lambda i: (0, 0)),
            pl.BlockSpec((1, 64), lambda i: (0, 0)),
        ],
        out_specs=pl.BlockSpec((TB, 64), lambda i: (i, 0)),
        out_shape=jax.ShapeDtypeStruct((B, 64), jnp.float32),
    )(pooled, b1.reshape(1, -1), W2, b2.reshape(1, -1), W3,
      b3.reshape(1, -1))


def kernel(semantic_tokens, table, W1, b1, W2, b2, W3, b3):
    tokens_2d = semantic_tokens.reshape(NW * N_CHUNKS, CHUNK)
    proj = _project(table, W1)
    pooled_sum = _sc_pool(proj, tokens_2d)
    return _head(pooled_sum, b1, W2, b2, W3, b3)


# speedup vs baseline: 1.2898x; 1.1711x over previous
import jax
import jax.numpy as jnp
import numpy as np
from jax import lax
from jax.experimental import pallas as pl
from jax.experimental.pallas import tpu as pltpu
from jax.experimental.pallas import tpu_sc as plsc

VOCAB = 100000
EMB = 512
B = 16384
L = 20
H1 = 256
HW = H1 // 2

NC = 2
NS = 16
NW = NC * NS
BPW = B // NW
CB = 4
CHUNK = CB * L
CB_OUT = 2 * CB
N_OUT = BPW // CB_OUT
N_CHUNKS = BPW // CB
LANES = 16
NGRP = HW // LANES

_HI_MASK = np.int32(-65536)
_RND = np.int32(0x8000)


def _project_body(t_ref, w1e_ref, w1o_ref, o_ref):
    dn = (((1,), (1,)), ((), ()))
    t16 = t_ref[...].astype(jnp.bfloat16)
    lo = lax.dot_general(t16, w1e_ref[...].astype(jnp.bfloat16), dn,
                         preferred_element_type=jnp.float32)
    hi = lax.dot_general(t16, w1o_ref[...].astype(jnp.bfloat16), dn,
                         preferred_element_type=jnp.float32)
    lo_b = lax.bitcast_convert_type(lo, jnp.int32)
    hi_b = lax.bitcast_convert_type(hi, jnp.int32)
    lo16 = lax.shift_right_logical(lo_b + _RND, 16)
    hi16 = lax.bitwise_and(hi_b + _RND, _HI_MASK)
    o_ref[...] = lax.bitwise_or(lo16, hi16)


def _project(table, W1e, W1o):
    RB = 1000
    return pl.pallas_call(
        _project_body,
        grid=(VOCAB // RB,),
        in_specs=[
            pl.BlockSpec((RB, EMB), lambda i: (i, 0)),
            pl.BlockSpec((HW, EMB), lambda i: (0, 0)),
            pl.BlockSpec((HW, EMB), lambda i: (0, 0)),
        ],
        out_specs=pl.BlockSpec((RB, HW), lambda i: (i, 0)),
        out_shape=jax.ShapeDtypeStruct((VOCAB, HW), jnp.int32),
    )(table, W1e, W1o)


def _sc_pool_body(table_hbm, tok_hbm, out_hbm,
                  idx_all, rows0, rows1, stage_v,
                  sem0, sem1):
    wid = lax.axis_index("s") * NC + lax.axis_index("c")
    out_base = wid * BPW

    row_bufs = [rows0, rows1]
    sems = [sem0, sem1]

    pltpu.sync_copy(tok_hbm.at[pl.ds(wid * N_CHUNKS, N_CHUNKS)], idx_all)

    def gather_start(g, buf):
        pltpu.async_copy(table_hbm.at[idx_all.at[g]], row_bufs[buf],
                         sems[buf])

    def gather_wait(g, buf):
        pltpu.make_async_copy(table_hbm.at[idx_all.at[g]], row_bufs[buf],
                              sems[buf]).wait()

    def pool_chunk(rows, stage_base):
        def col_body(c, carry):
            sl = pl.ds(c * LANES, LANES)
            for b in range(CB):
                v = rows[b * L + 0, sl]
                s_lo = lax.bitcast_convert_type(lax.shift_left(v, 16),
                                                jnp.float32)
                s_hi = lax.bitcast_convert_type(lax.bitwise_and(v, _HI_MASK),
                                                jnp.float32)
                for r in range(1, L):
                    v = rows[b * L + r, sl]
                    s_lo = s_lo + lax.bitcast_convert_type(
                        lax.shift_left(v, 16), jnp.float32)
                    s_hi = s_hi + lax.bitcast_convert_type(
                        lax.bitwise_and(v, _HI_MASK), jnp.float32)
                lo_b = lax.bitcast_convert_type(s_lo, jnp.int32)
                hi_b = lax.bitcast_convert_type(s_hi, jnp.int32)
                packed = lax.bitwise_or(
                    lax.shift_right_logical(lo_b + _RND, 16),
                    lax.bitwise_and(hi_b + _RND, _HI_MASK))
                stage_v[stage_base + b, sl] = packed
            return carry

        lax.fori_loop(0, NGRP, col_body, 0)

    gather_start(0, 0)
    gather_start(1, 1)

    def iter_body(i, carry):
        gather_wait(2 * i, 0)
        pool_chunk(rows0, 0)

        @pl.when(i < N_OUT - 1)
        def _():
            gather_start(2 * i + 2, 0)

        gather_wait(2 * i + 1, 1)
        pool_chunk(rows1, CB)

        @pl.when(i < N_OUT - 1)
        def _():
            gather_start(2 * i + 3, 1)

        pltpu.sync_copy(stage_v,
                        out_hbm.at[pl.ds(out_base + i * CB_OUT, CB_OUT)])
        return carry

    lax.fori_loop(0, N_OUT, iter_body, 0)


def _sc_pool(proj, tokens_2d):
    pool = pl.kernel(
        _sc_pool_body,
        out_type=jax.ShapeDtypeStruct((B, HW), jnp.int32),
        mesh=plsc.VectorSubcoreMesh(core_axis_name="c", subcore_axis_name="s"),
        scratch_types=[
            pltpu.VMEM((N_CHUNKS, CHUNK), jnp.int32),
            pltpu.VMEM((CHUNK, HW), jnp.int32),
            pltpu.VMEM((CHUNK, HW), jnp.int32),
            pltpu.VMEM((CB_OUT, HW), jnp.int32),
            pltpu.SemaphoreType.DMA,
            pltpu.SemaphoreType.DMA,
        ],
    )
    return pool(proj, tokens_2d)


def _head_body(x_ref, b1_ref, w2_ref, b2_ref, w3_ref, b3_ref, o_ref):
    dn = (((1,), (1,)), ((), ()))
    v = x_ref[...]
    x_even = lax.bitcast_convert_type(lax.shift_left(v, 16), jnp.float32)
    x_odd = lax.bitcast_convert_type(lax.bitwise_and(v, _HI_MASK),
                                     jnp.float32)
    x = jnp.concatenate([x_even, x_odd], axis=1)
    h = jnp.maximum(x * (1.0 / L) + b1_ref[...], 0.0)
    h = lax.dot_general(h, w2_ref[...], dn, preferred_element_type=jnp.float32)
    h = jnp.maximum(h + b2_ref[...], 0.0)
    o = lax.dot_general(h, w3_ref[...], dn, preferred_element_type=jnp.float32)
    o_ref[...] = o + b3_ref[...]


def _head(pooled, b1p, W2p, b2, W3, b3):
    TB = 1024
    return pl.pallas_call(
        _head_body,
        grid=(B // TB,),
        in_specs=[
            pl.BlockSpec((TB, HW), lambda i: (i, 0)),
            pl.BlockSpec((1, H1), lambda i: (0, 0)),
            pl.BlockSpec((128, H1), lambda i: (0, 0)),
            pl.BlockSpec((1, 128), lambda i: (0, 0)),
            pl.BlockSpec((64, 128), lambda i: (0, 0)),
            pl.BlockSpec((1, 64), lambda i: (0, 0)),
        ],
        out_specs=pl.BlockSpec((TB, 64), lambda i: (i, 0)),
        out_shape=jax.ShapeDtypeStruct((B, 64), jnp.float32),
    )(pooled, b1p.reshape(1, -1), W2p, b2.reshape(1, -1), W3,
      b3.reshape(1, -1))


def kernel(semantic_tokens, table, W1, b1, W2, b2, W3, b3):
    tokens_2d = semantic_tokens.reshape(NW * N_CHUNKS, CHUNK)
    W1e, W1o = W1[0::2, :], W1[1::2, :]
    b1p = jnp.concatenate([b1[0::2], b1[1::2]])
    W2p = jnp.concatenate([W2[:, 0::2], W2[:, 1::2]], axis=1)
    proj = _project(table, W1e, W1o)
    pooled_sum = _sc_pool(proj, tokens_2d)
    return _head(pooled_sum, b1p, W2p, b2, W3, b3)


# speedup vs baseline: 1.4697x; 1.1395x over previous
import jax
import jax.numpy as jnp
import numpy as np
from jax import lax
from jax.experimental import pallas as pl
from jax.experimental.pallas import tpu as pltpu
from jax.experimental.pallas import tpu_sc as plsc

VOCAB = 100000
EMB = 512
B = 16384
L = 20
H1 = 256
HW = H1 // 2

NC = 2
NS = 16
NW = NC * NS
BPW = B // NW
CB = 4
CHUNK = CB * L
CB_OUT = 2 * CB
N_OUT = BPW // CB_OUT
N_CHUNKS = BPW // CB
LANES = 16
NGRP = HW // LANES

_HI_MASK = np.int32(-65536)
_RND = np.int32(0x8000)


def _project_body(t_ref, w1e_ref, w1o_ref, o_ref):
    dn = (((1,), (1,)), ((), ()))
    t16 = t_ref[...].astype(jnp.bfloat16)
    lo = lax.dot_general(t16, w1e_ref[...].astype(jnp.bfloat16), dn,
                         preferred_element_type=jnp.float32)
    hi = lax.dot_general(t16, w1o_ref[...].astype(jnp.bfloat16), dn,
                         preferred_element_type=jnp.float32)
    lo_b = lax.bitcast_convert_type(lo, jnp.int32)
    hi_b = lax.bitcast_convert_type(hi, jnp.int32)
    lo16 = lax.shift_right_logical(lo_b + _RND, 16)
    hi16 = lax.bitwise_and(hi_b + _RND, _HI_MASK)
    o_ref[...] = lax.bitwise_or(lo16, hi16)


def _project(table, W1e, W1o):
    RB = 2000
    return pl.pallas_call(
        _project_body,
        grid=(VOCAB // RB,),
        in_specs=[
            pl.BlockSpec((RB, EMB), lambda i: (i, 0)),
            pl.BlockSpec((HW, EMB), lambda i: (0, 0)),
            pl.BlockSpec((HW, EMB), lambda i: (0, 0)),
        ],
        out_specs=pl.BlockSpec((RB, HW), lambda i: (i, 0)),
        out_shape=jax.ShapeDtypeStruct((VOCAB, HW), jnp.int32),
    )(table, W1e, W1o)


def _sc_pool_body(table_hbm, tok_hbm, out_hbm,
                  idx_all, rows0, rows1, stage_v,
                  sem0, sem1):
    wid = lax.axis_index("s") * NC + lax.axis_index("c")
    out_base = wid * BPW

    row_bufs = [rows0, rows1]
    sems = [sem0, sem1]

    pltpu.sync_copy(tok_hbm.at[pl.ds(wid * N_CHUNKS, N_CHUNKS)], idx_all)

    def gather_start(g, buf):
        pltpu.async_copy(table_hbm.at[idx_all.at[g]], row_bufs[buf],
                         sems[buf])

    def gather_wait(g, buf):
        pltpu.make_async_copy(table_hbm.at[idx_all.at[g]], row_bufs[buf],
                              sems[buf]).wait()

    def pool_chunk(rows, stage_base):
        def col_body(c, carry):
            sl = pl.ds(c * LANES, LANES)
            for b in range(CB):
                v = rows[b * L + 0, sl]
                s_lo = lax.bitcast_convert_type(lax.shift_left(v, 16),
                                                jnp.float32)
                s_hi = lax.bitcast_convert_type(v, jnp.float32)
                for r in range(1, L):
                    v = rows[b * L + r, sl]
                    s_lo = s_lo + lax.bitcast_convert_type(
                        lax.shift_left(v, 16), jnp.float32)
                    s_hi = s_hi + lax.bitcast_convert_type(v, jnp.float32)
                lo_b = lax.bitcast_convert_type(s_lo, jnp.int32)
                hi_b = lax.bitcast_convert_type(s_hi, jnp.int32)
                packed = lax.bitwise_or(
                    lax.shift_right_logical(lo_b + _RND, 16),
                    lax.bitwise_and(hi_b + _RND, _HI_MASK))
                stage_v[stage_base + b, sl] = packed
            return carry

        lax.fori_loop(0, NGRP, col_body, 0)

    gather_start(0, 0)
    gather_start(1, 1)

    def iter_body(i, carry):
        gather_wait(2 * i, 0)
        pool_chunk(rows0, 0)

        @pl.when(i < N_OUT - 1)
        def _():
            gather_start(2 * i + 2, 0)

        gather_wait(2 * i + 1, 1)
        pool_chunk(rows1, CB)

        @pl.when(i < N_OUT - 1)
        def _():
            gather_start(2 * i + 3, 1)

        pltpu.sync_copy(stage_v,
                        out_hbm.at[pl.ds(out_base + i * CB_OUT, CB_OUT)])
        return carry

    lax.fori_loop(0, N_OUT, iter_body, 0)


def _sc_pool(proj, tokens_2d):
    pool = pl.kernel(
        _sc_pool_body,
        out_type=jax.ShapeDtypeStruct((B, HW), jnp.int32),
        mesh=plsc.VectorSubcoreMesh(core_axis_name="c", subcore_axis_name="s"),
        scratch_types=[
            pltpu.VMEM((N_CHUNKS, CHUNK), jnp.int32),
            pltpu.VMEM((CHUNK, HW), jnp.int32),
            pltpu.VMEM((CHUNK, HW), jnp.int32),
            pltpu.VMEM((CB_OUT, HW), jnp.int32),
            pltpu.SemaphoreType.DMA,
            pltpu.SemaphoreType.DMA,
        ],
    )
    return pool(proj, tokens_2d)


def _head_body(x_ref, b1_ref, w2_ref, b2_ref, w3_ref, b3_ref, o_ref):
    dn = (((1,), (1,)), ((), ()))
    v = x_ref[...]
    x_even = lax.bitcast_convert_type(lax.shift_left(v, 16), jnp.float32)
    x_odd = lax.bitcast_convert_type(lax.bitwise_and(v, _HI_MASK),
                                     jnp.float32)
    x = jnp.concatenate([x_even, x_odd], axis=1)
    h = jnp.maximum(x * (1.0 / L) + b1_ref[...], 0.0)
    h = lax.dot_general(h, w2_ref[...], dn, preferred_element_type=jnp.float32)
    h = jnp.maximum(h + b2_ref[...], 0.0)
    o = lax.dot_general(h, w3_ref[...], dn, preferred_element_type=jnp.float32)
    o_ref[...] = o + b3_ref[...]


def _head(pooled, b1p, W2p, b2, W3, b3):
    TB = 1024
    return pl.pallas_call(
        _head_body,
        grid=(B // TB,),
        in_specs=[
            pl.BlockSpec((TB, HW), lambda i: (i, 0)),
            pl.BlockSpec((1, H1), lambda i: (0, 0)),
            pl.BlockSpec((128, H1), lambda i: (0, 0)),
            pl.BlockSpec((1, 128), lambda i: (0, 0)),
            pl.BlockSpec((64, 128), lambda i: (0, 0)),
            pl.BlockSpec((1, 64), lambda i: (0, 0)),
        ],
        out_specs=pl.BlockSpec((TB, 64), lambda i: (i, 0)),
        out_shape=jax.ShapeDtypeStruct((B, 64), jnp.float32),
    )(pooled, b1p.reshape(1, -1), W2p, b2.reshape(1, -1), W3,
      b3.reshape(1, -1))


def kernel(semantic_tokens, table, W1, b1, W2, b2, W3, b3):
    tokens_2d = semantic_tokens.reshape(NW * N_CHUNKS, CHUNK)
    W1e, W1o = W1[0::2, :], W1[1::2, :]
    b1p = jnp.concatenate([b1[0::2], b1[1::2]])
    W2p = jnp.concatenate([W2[:, 0::2], W2[:, 1::2]], axis=1)
    proj = _project(table, W1e, W1o)
    pooled_sum = _sc_pool(proj, tokens_2d)
    return _head(pooled_sum, b1p, W2p, b2, W3, b3)


# speedup vs baseline: 1.5990x; 1.0880x over previous
import jax
import jax.numpy as jnp
import numpy as np
from jax import lax
from jax.experimental import pallas as pl
from jax.experimental.pallas import tpu as pltpu
from jax.experimental.pallas import tpu_sc as plsc

VOCAB = 100000
EMB = 512
B = 16384
L = 20
H1 = 256
HW = H1 // 2

NC = 2
NS = 16
NW = NC * NS
BPW = B // NW
CB = 4
CHUNK = CB * L
NBUF = 4
CB_OUT = NBUF * CB
N_OUT = BPW // CB_OUT
N_CHUNKS = BPW // CB
LANES = 16
NGRP = HW // LANES

_HI_MASK = np.int32(-65536)
_RND = np.int32(0x8000)


def _project_body(t_ref, w1e_ref, w1o_ref, o_ref):
    dn = (((1,), (1,)), ((), ()))
    t16 = t_ref[...].astype(jnp.bfloat16)
    lo = lax.dot_general(t16, w1e_ref[...].astype(jnp.bfloat16), dn,
                         preferred_element_type=jnp.float32)
    hi = lax.dot_general(t16, w1o_ref[...].astype(jnp.bfloat16), dn,
                         preferred_element_type=jnp.float32)
    lo_b = lax.bitcast_convert_type(lo, jnp.int32)
    hi_b = lax.bitcast_convert_type(hi, jnp.int32)
    lo16 = lax.shift_right_logical(lo_b + _RND, 16)
    hi16 = lax.bitwise_and(hi_b + _RND, _HI_MASK)
    o_ref[...] = lax.bitwise_or(lo16, hi16)


def _project(table, W1e, W1o):
    RB = 2000
    return pl.pallas_call(
        _project_body,
        grid=(VOCAB // RB,),
        in_specs=[
            pl.BlockSpec((RB, EMB), lambda i: (i, 0)),
            pl.BlockSpec((HW, EMB), lambda i: (0, 0)),
            pl.BlockSpec((HW, EMB), lambda i: (0, 0)),
        ],
        out_specs=pl.BlockSpec((RB, HW), lambda i: (i, 0)),
        out_shape=jax.ShapeDtypeStruct((VOCAB, HW), jnp.int32),
    )(table, W1e, W1o)


def _sc_pool_body(table_hbm, tok_hbm, out_hbm,
                  idx_all, rows0, rows1, rows2, rows3, stage_v,
                  sem0, sem1, sem2, sem3):
    wid = lax.axis_index("s") * NC + lax.axis_index("c")
    out_base = wid * BPW

    row_bufs = [rows0, rows1, rows2, rows3]
    sems = [sem0, sem1, sem2, sem3]

    pltpu.sync_copy(tok_hbm.at[pl.ds(wid * N_CHUNKS, N_CHUNKS)], idx_all)

    def gather_start(g, buf):
        pltpu.async_copy(table_hbm.at[idx_all.at[g]], row_bufs[buf],
                         sems[buf])

    def gather_wait(g, buf):
        pltpu.make_async_copy(table_hbm.at[idx_all.at[g]], row_bufs[buf],
                              sems[buf]).wait()

    def pool_chunk(rows, stage_base):
        def col_body(c, carry):
            sl = pl.ds(c * LANES, LANES)
            for b in range(CB):
                v = rows[b * L + 0, sl]
                s_lo = lax.bitcast_convert_type(lax.shift_left(v, 16),
                                                jnp.float32)
                s_hi = lax.bitcast_convert_type(v, jnp.float32)
                for r in range(1, L):
                    v = rows[b * L + r, sl]
                    s_lo = s_lo + lax.bitcast_convert_type(
                        lax.shift_left(v, 16), jnp.float32)
                    s_hi = s_hi + lax.bitcast_convert_type(v, jnp.float32)
                lo_b = lax.bitcast_convert_type(s_lo, jnp.int32)
                hi_b = lax.bitcast_convert_type(s_hi, jnp.int32)
                packed = lax.bitwise_or(
                    lax.shift_right_logical(lo_b + _RND, 16),
                    lax.bitwise_and(hi_b + _RND, _HI_MASK))
                stage_v[stage_base + b, sl] = packed
            return carry

        lax.fori_loop(0, NGRP, col_body, 0)

    for j in range(NBUF):
        gather_start(j, j)

    def iter_body(i, carry):
        for j in range(NBUF):
            gather_wait(NBUF * i + j, j)
            pool_chunk(row_bufs[j], CB * j)

            @pl.when(i < N_OUT - 1)
            def _():
                gather_start(NBUF * i + j + NBUF, j)

        pltpu.sync_copy(stage_v,
                        out_hbm.at[pl.ds(out_base + i * CB_OUT, CB_OUT)])
        return carry

    lax.fori_loop(0, N_OUT, iter_body, 0)


def _sc_pool(proj, tokens_2d):
    pool = pl.kernel(
        _sc_pool_body,
        out_type=jax.ShapeDtypeStruct((B, HW), jnp.int32),
        mesh=plsc.VectorSubcoreMesh(core_axis_name="c", subcore_axis_name="s"),
        scratch_types=[
            pltpu.VMEM((N_CHUNKS, CHUNK), jnp.int32),
            pltpu.VMEM((CHUNK, HW), jnp.int32),
            pltpu.VMEM((CHUNK, HW), jnp.int32),
            pltpu.VMEM((CHUNK, HW), jnp.int32),
            pltpu.VMEM((CHUNK, HW), jnp.int32),
            pltpu.VMEM((CB_OUT, HW), jnp.int32),
            pltpu.SemaphoreType.DMA,
            pltpu.SemaphoreType.DMA,
            pltpu.SemaphoreType.DMA,
            pltpu.SemaphoreType.DMA,
        ],
    )
    return pool(proj, tokens_2d)


def _head_body(x_ref, b1_ref, w2_ref, b2_ref, w3_ref, b3_ref, o_ref):
    dn = (((1,), (1,)), ((), ()))
    v = x_ref[...]
    x_even = lax.bitcast_convert_type(lax.shift_left(v, 16), jnp.float32)
    x_odd = lax.bitcast_convert_type(lax.bitwise_and(v, _HI_MASK),
                                     jnp.float32)
    x = jnp.concatenate([x_even, x_odd], axis=1)
    h = jnp.maximum(x * (1.0 / L) + b1_ref[...], 0.0)
    h = lax.dot_general(h, w2_ref[...], dn, preferred_element_type=jnp.float32)
    h = jnp.maximum(h + b2_ref[...], 0.0)
    o = lax.dot_general(h, w3_ref[...], dn, preferred_element_type=jnp.float32)
    o_ref[...] = o + b3_ref[...]


def _head(pooled, b1p, W2p, b2, W3, b3):
    TB = 1024
    return pl.pallas_call(
        _head_body,
        grid=(B // TB,),
        in_specs=[
            pl.BlockSpec((TB, HW), lambda i: (i, 0)),
            pl.BlockSpec((1, H1), lambda i: (0, 0)),
            pl.BlockSpec((128, H1), lambda i: (0, 0)),
            pl.BlockSpec((1, 128), lambda i: (0, 0)),
            pl.BlockSpec((64, 128), lambda i: (0, 0)),
            pl.BlockSpec((1, 64), lambda i: (0, 0)),
        ],
        out_specs=pl.BlockSpec((TB, 64), lambda i: (i, 0)),
        out_shape=jax.ShapeDtypeStruct((B, 64), jnp.float32),
    )(pooled, b1p.reshape(1, -1), W2p, b2.reshape(1, -1), W3,
      b3.reshape(1, -1))


def kernel(semantic_tokens, table, W1, b1, W2, b2, W3, b3):
    tokens_2d = semantic_tokens.reshape(NW * N_CHUNKS, CHUNK)
    W1e, W1o = W1[0::2, :], W1[1::2, :]
    b1p = jnp.concatenate([b1[0::2], b1[1::2]])
    W2p = jnp.concatenate([W2[:, 0::2], W2[:, 1::2]], axis=1)
    proj = _project(table, W1e, W1o)
    pooled_sum = _sc_pool(proj, tokens_2d)
    return _head(pooled_sum, b1p, W2p, b2, W3, b3)


# speedup vs baseline: 1.6691x; 1.0438x over previous
import jax
import jax.numpy as jnp
import numpy as np
from jax import lax
from jax.experimental import pallas as pl
from jax.experimental.pallas import tpu as pltpu
from jax.experimental.pallas import tpu_sc as plsc

VOCAB = 100000
EMB = 512
B = 16384
L = 20
H1 = 256
HW = H1 // 2

NC = 2
NS = 16
NW = NC * NS
BPW = B // NW
CB = 4
CHUNK = CB * L
NBUF = 8
CB_OUT = NBUF * CB
N_OUT = BPW // CB_OUT
N_CHUNKS = BPW // CB
LANES = 16
NGRP = HW // LANES

_HI_MASK = np.int32(-65536)
_RND = np.int32(0x8000)


def _project_body(t_ref, w1e_ref, w1o_ref, o_ref):
    dn = (((1,), (1,)), ((), ()))
    t16 = t_ref[...].astype(jnp.bfloat16)
    lo = lax.dot_general(t16, w1e_ref[...].astype(jnp.bfloat16), dn,
                         preferred_element_type=jnp.float32)
    hi = lax.dot_general(t16, w1o_ref[...].astype(jnp.bfloat16), dn,
                         preferred_element_type=jnp.float32)
    lo_b = lax.bitcast_convert_type(lo, jnp.int32)
    hi_b = lax.bitcast_convert_type(hi, jnp.int32)
    lo16 = lax.shift_right_logical(lo_b + _RND, 16)
    hi16 = lax.bitwise_and(hi_b + _RND, _HI_MASK)
    o_ref[...] = lax.bitwise_or(lo16, hi16)


def _project(table, W1e, W1o):
    RB = 4000
    return pl.pallas_call(
        _project_body,
        grid=(VOCAB // RB,),
        in_specs=[
            pl.BlockSpec((RB, EMB), lambda i: (i, 0)),
            pl.BlockSpec((HW, EMB), lambda i: (0, 0)),
            pl.BlockSpec((HW, EMB), lambda i: (0, 0)),
        ],
        out_specs=pl.BlockSpec((RB, HW), lambda i: (i, 0)),
        out_shape=jax.ShapeDtypeStruct((VOCAB, HW), jnp.int32),
    )(table, W1e, W1o)


def _sc_pool_body(table_hbm, tok_hbm, out_hbm,
                  idx_all, rows0, rows1, rows2, rows3,
                  rows4, rows5, rows6, rows7, stage_v,
                  sem0, sem1, sem2, sem3, sem4, sem5, sem6, sem7):
    wid = lax.axis_index("s") * NC + lax.axis_index("c")
    out_base = wid * BPW

    row_bufs = [rows0, rows1, rows2, rows3, rows4, rows5, rows6, rows7]
    sems = [sem0, sem1, sem2, sem3, sem4, sem5, sem6, sem7]

    pltpu.sync_copy(tok_hbm.at[pl.ds(wid * N_CHUNKS, N_CHUNKS)], idx_all)

    def gather_start(g, buf):
        pltpu.async_copy(table_hbm.at[idx_all.at[g]], row_bufs[buf],
                         sems[buf])

    def gather_wait(g, buf):
        pltpu.make_async_copy(table_hbm.at[idx_all.at[g]], row_bufs[buf],
                              sems[buf]).wait()

    def pool_chunk(rows, stage_base):
        def col_body(c, carry):
            sl = pl.ds(c * LANES, LANES)
            for b in range(CB):
                v = rows[b * L + 0, sl]
                s_lo = lax.bitcast_convert_type(lax.shift_left(v, 16),
                                                jnp.float32)
                s_hi = lax.bitcast_convert_type(v, jnp.float32)
                for r in range(1, L):
                    v = rows[b * L + r, sl]
                    s_lo = s_lo + lax.bitcast_convert_type(
                        lax.shift_left(v, 16), jnp.float32)
                    s_hi = s_hi + lax.bitcast_convert_type(v, jnp.float32)
                lo_b = lax.bitcast_convert_type(s_lo, jnp.int32)
                hi_b = lax.bitcast_convert_type(s_hi, jnp.int32)
                packed = lax.bitwise_or(
                    lax.shift_right_logical(lo_b + _RND, 16),
                    lax.bitwise_and(hi_b + _RND, _HI_MASK))
                stage_v[stage_base + b, sl] = packed
            return carry

        lax.fori_loop(0, NGRP, col_body, 0)

    for j in range(NBUF):
        gather_start(j, j)

    def iter_body(i, carry):
        for j in range(NBUF):
            gather_wait(NBUF * i + j, j)
            pool_chunk(row_bufs[j], CB * j)

            @pl.when(i < N_OUT - 1)
            def _():
                gather_start(NBUF * i + j + NBUF, j)

        pltpu.sync_copy(stage_v,
                        out_hbm.at[pl.ds(out_base + i * CB_OUT, CB_OUT)])
        return carry

    lax.fori_loop(0, N_OUT, iter_body, 0)


def _sc_pool(proj, tokens_2d):
    pool = pl.kernel(
        _sc_pool_body,
        out_type=jax.ShapeDtypeStruct((B, HW), jnp.int32),
        mesh=plsc.VectorSubcoreMesh(core_axis_name="c", subcore_axis_name="s"),
        scratch_types=(
            [pltpu.VMEM((N_CHUNKS, CHUNK), jnp.int32)]
            + [pltpu.VMEM((CHUNK, HW), jnp.int32)] * NBUF
            + [pltpu.VMEM((CB_OUT, HW), jnp.int32)]
            + [pltpu.SemaphoreType.DMA] * NBUF
        ),
    )
    return pool(proj, tokens_2d)


def _head_body(x_ref, b1_ref, w2_ref, b2_ref, w3_ref, b3_ref, o_ref):
    dn = (((1,), (1,)), ((), ()))
    v = x_ref[...]
    x_even = lax.bitcast_convert_type(lax.shift_left(v, 16), jnp.float32)
    x_odd = lax.bitcast_convert_type(lax.bitwise_and(v, _HI_MASK),
                                     jnp.float32)
    x = jnp.concatenate([x_even, x_odd], axis=1)
    h = jnp.maximum(x * (1.0 / L) + b1_ref[...], 0.0)
    h = lax.dot_general(h, w2_ref[...], dn, preferred_element_type=jnp.float32)
    h = jnp.maximum(h + b2_ref[...], 0.0)
    o = lax.dot_general(h, w3_ref[...], dn, preferred_element_type=jnp.float32)
    o_ref[...] = o + b3_ref[...]


def _head(pooled, b1p, W2p, b2, W3, b3):
    TB = 1024
    return pl.pallas_call(
        _head_body,
        grid=(B // TB,),
        in_specs=[
            pl.BlockSpec((TB, HW), lambda i: (i, 0)),
            pl.BlockSpec((1, H1), lambda i: (0, 0)),
            pl.BlockSpec((128, H1), lambda i: (0, 0)),
            pl.BlockSpec((1, 128), lambda i: (0, 0)),
            pl.BlockSpec((64, 128), lambda i: (0, 0)),
            pl.BlockSpec((1, 64), lambda i: (0, 0)),
        ],
        out_specs=pl.BlockSpec((TB, 64), lambda i: (i, 0)),
        out_shape=jax.ShapeDtypeStruct((B, 64), jnp.float32),
    )(pooled, b1p.reshape(1, -1), W2p, b2.reshape(1, -1), W3,
      b3.reshape(1, -1))


def kernel(semantic_tokens, table, W1, b1, W2, b2, W3, b3):
    tokens_2d = semantic_tokens.reshape(NW * N_CHUNKS, CHUNK)
    W1e, W1o = W1[0::2, :], W1[1::2, :]
    b1p = jnp.concatenate([b1[0::2], b1[1::2]])
    W2p = jnp.concatenate([W2[:, 0::2], W2[:, 1::2]], axis=1)
    proj = _project(table, W1e, W1o)
    pooled_sum = _sc_pool(proj, tokens_2d)
    return _head(pooled_sum, b1p, W2p, b2, W3, b3)
